# baseline (speedup 1.0000x reference)
import functools

import jax
import jax.numpy as jnp
from jax import lax
from jax.experimental import pallas as pl
from jax.experimental.pallas import tpu as pltpu
from jax.experimental.pallas import tpu_sc as plsc

_D = 64
_SCALE = 8.0

_NC = 2
_NS = 16
_NW = _NC * _NS

_CHUNK = 1024


@functools.partial(jax.jit, static_argnames=("n_rows",))
def _embed_lookup(table, idx, n_rows):
    b_per_w = n_rows // _NW
    n_chunks = b_per_w // _CHUNK
    mesh = plsc.VectorSubcoreMesh(core_axis_name="c", subcore_axis_name="s")

    @functools.partial(
        pl.kernel,
        out_type=jax.ShapeDtypeStruct((n_rows, _D), jnp.float32),
        mesh=mesh,
        scratch_types=[
            pltpu.VMEM((_CHUNK,), jnp.int32),
            pltpu.VMEM((_CHUNK, _D), jnp.float32),
            pltpu.SemaphoreType.DMA,
        ],
        compiler_params=pltpu.CompilerParams(use_tc_tiling_on_sc=False),
    )
    def k(table_hbm, idx_hbm, out_hbm, idx_v, rows_v, sem):
        wid = lax.axis_index("s") * _NC + lax.axis_index("c")
        base = wid * b_per_w

        def chunk_body(g, carry):
            off = base + g * _CHUNK
            pltpu.sync_copy(idx_hbm.at[pl.ds(off, _CHUNK)], idx_v)
            pltpu.async_copy(table_hbm.at[idx_v], rows_v, sem).wait()

            def scale_row(r, c2):
                for j in range(_D // 16):
                    sl = pl.ds(j * 16, 16)
                    rows_v[r, sl] = rows_v[r, sl] * _SCALE
                return c2

            lax.fori_loop(0, _CHUNK, scale_row, 0)
            pltpu.sync_copy(rows_v, out_hbm.at[pl.ds(off, _CHUNK)])
            return carry

        lax.fori_loop(0, n_chunks, chunk_body, 0)

    return k(table, idx)


def kernel(x, table):
    idx = x.reshape(-1).astype(jnp.int32)
    out = _embed_lookup(table, idx, idx.shape[0])
    return out.reshape(x.shape + (table.shape[1],))


# speedup vs baseline: 1.1004x; 1.1004x over previous
import functools

import jax
import jax.numpy as jnp
from jax import lax
from jax.experimental import pallas as pl
from jax.experimental.pallas import tpu as pltpu
from jax.experimental.pallas import tpu_sc as plsc

_D = 64
_SCALE = 8.0

_NC = 2
_NS = 16
_NW = _NC * _NS

_CHUNK = 800
_UNROLL = 8


@functools.partial(jax.jit, static_argnames=("n_rows",))
def _embed_lookup(table, idx, n_rows):
    b_per_w = n_rows // _NW
    n_chunks = b_per_w // _CHUNK
    n_pairs = n_chunks // 2
    mesh = plsc.VectorSubcoreMesh(core_axis_name="c", subcore_axis_name="s")

    @functools.partial(
        pl.kernel,
        out_type=jax.ShapeDtypeStruct((n_rows, _D), jnp.float32),
        mesh=mesh,
        scratch_types=[
            pltpu.VMEM((_CHUNK,), jnp.int32),
            pltpu.VMEM((_CHUNK,), jnp.int32),
            pltpu.VMEM((_CHUNK, _D), jnp.float32),
            pltpu.VMEM((_CHUNK, _D), jnp.float32),
            pltpu.SemaphoreType.DMA,
            pltpu.SemaphoreType.DMA,
        ],
        compiler_params=pltpu.CompilerParams(use_tc_tiling_on_sc=False),
    )
    def k(table_hbm, idx_hbm, out_hbm, idx_a, idx_b, rows_a, rows_b,
          sem_a, sem_b):
        wid = lax.axis_index("s") * _NC + lax.axis_index("c")
        base = wid * b_per_w

        def scale(rows):
            def body(i, carry):
                r0 = i * _UNROLL
                for u in range(_UNROLL):
                    for j in range(_D // 16):
                        sl = pl.ds(j * 16, 16)
                        rows[r0 + u, sl] = rows[r0 + u, sl] * _SCALE
                return carry
            lax.fori_loop(0, _CHUNK // _UNROLL, body, 0)

        pltpu.sync_copy(idx_hbm.at[pl.ds(base, _CHUNK)], idx_a)
        pltpu.async_copy(table_hbm.at[idx_a], rows_a, sem_a)

        def pair_body(h, carry):
            g0 = 2 * h
            off0 = base + g0 * _CHUNK
            off1 = off0 + _CHUNK

            pltpu.sync_copy(idx_hbm.at[pl.ds(off1, _CHUNK)], idx_b)
            pltpu.async_copy(table_hbm.at[idx_b], rows_b, sem_b)

            pltpu.make_async_copy(table_hbm.at[idx_a], rows_a, sem_a).wait()
            scale(rows_a)
            pltpu.sync_copy(rows_a, out_hbm.at[pl.ds(off0, _CHUNK)])

            @pl.when(h + 1 < n_pairs)
            def _():
                off2 = off1 + _CHUNK
                pltpu.sync_copy(idx_hbm.at[pl.ds(off2, _CHUNK)], idx_a)
                pltpu.async_copy(table_hbm.at[idx_a], rows_a, sem_a)

            pltpu.make_async_copy(table_hbm.at[idx_b], rows_b, sem_b).wait()
            scale(rows_b)
            pltpu.sync_copy(rows_b, out_hbm.at[pl.ds(off1, _CHUNK)])
            return carry

        lax.fori_loop(0, n_pairs, pair_body, 0)

    return k(table, idx)


def kernel(x, table):
    idx = x.reshape(-1).astype(jnp.int32)
    out = _embed_lookup(table, idx, idx.shape[0])
    return out.reshape(x.shape + (table.shape[1],))
